# Initial kernel scaffold; baseline (speedup 1.0000x reference)
import jax
import jax.numpy as jnp
from jax.experimental import pallas as pl


def kernel(x, emb):
    raise NotImplementedError("write your pallas kernel here")


# speedup vs baseline: 3.1922x; 3.1922x over previous
import jax
import jax.numpy as jnp
from jax.experimental import pallas as pl


def _copy_block(emb_ref, out_ref):
    out_ref[0, :, :] = emb_ref[:, :]


def kernel(x, emb):
    T = x.shape[1]
    D = emb.shape[1]
    block = 1024
    assert T % block == 0
    out = pl.pallas_call(
        _copy_block,
        grid=(T // block,),
        in_specs=[pl.BlockSpec((block, D), lambda i: (i, 0))],
        out_specs=pl.BlockSpec((1, block, D), lambda i: (0, i, 0)),
        out_shape=jax.ShapeDtypeStruct((1, T, D), emb.dtype),
    )(emb[:T])
    return out
